# baseline (speedup 1.0000x reference)
import functools

import jax
import jax.numpy as jnp
from jax import lax
from jax.experimental import pallas as pl
from jax.experimental.pallas import tpu as pltpu
from jax.experimental.pallas import tpu_sc as plsc

DIM = 64
LANE = 128


@functools.cache
def _build(V, B, T, NC, NS):
    NW = NC * NS
    JT = B // LANE
    assert JT == NW
    IB = T // 8
    mesh = plsc.VectorSubcoreMesh(core_axis_name="c", subcore_axis_name="s")

    @functools.partial(
        pl.kernel,
        mesh=mesh,
        out_type=jax.ShapeDtypeStruct((T, DIM // 8, JT, 8, LANE), jnp.float32),
        scratch_types=[
            pltpu.VMEM((IB, 8, LANE), jnp.int32),
            pltpu.VMEM((LANE, DIM), jnp.float32),
            pltpu.VMEM((LANE, DIM), jnp.float32),
            pltpu.VMEM((DIM // 8, 8, LANE), jnp.float32),
            pltpu.SemaphoreType.DMA,
            pltpu.SemaphoreType.DMA,
        ],
        compiler_params=pltpu.CompilerParams(
            use_tc_tiling_on_sc=False, needs_layout_passes=False
        ),
    )
    def k(xl_hbm, w2_hbm, out_hbm, idx_v, g0, g1, t_v, sem0, sem1):
        w = lax.axis_index("s") * NC + lax.axis_index("c")
        pltpu.sync_copy(xl_hbm.at[:, w], idx_v)
        lanes16 = lax.iota(jnp.int32, 16)
        c_vecs = [lanes16 + 16 * c0 for c0 in range(8)]
        zero16 = lanes16 * 0

        gbufs = (g0, g1)
        sems = (sem0, sem1)
        pltpu.async_copy(w2_hbm.at[idx_v.at[0, 0]], g0, sem0)

        def chunk(t, g, par):
            i = t // 8
            r = lax.rem(t, 8)
            pltpu.make_async_copy(
                w2_hbm.at[idx_v.at[i, r]], gbufs[par], sems[par]
            ).wait()

            t1 = t + 1
            i1 = t1 // 8
            r1 = lax.rem(t1, 8)

            def issue_next():
                pltpu.async_copy(
                    w2_hbm.at[idx_v.at[i1, r1]], gbufs[1 - par], sems[1 - par]
                )

            if par == 0:
                issue_next()
            else:
                pl.when(g < (T // 2) - 1)(issue_next)

            def tbody(d, c2):
                ih = d // 8
                rh = lax.rem(d, 8)
                d_vec = zero16 + d
                for c0 in range(8):
                    vec = plsc.load_gather(gbufs[par], [c_vecs[c0], d_vec])
                    t_v[ih, rh, pl.ds(16 * c0, 16)] = vec
                return c2

            lax.fori_loop(0, DIM, tbody, 0)
            pltpu.sync_copy(t_v, out_hbm.at[t, :, w])

        def gbody(g, carry):
            chunk(2 * g, g, 0)
            chunk(2 * g + 1, g, 1)
            return carry

        lax.fori_loop(0, T // 2, gbody, 0)

    return k


def kernel(x, emb_weight):
    B, T = x.shape
    V = emb_weight.shape[0]
    NC, NS = 2, 16
    xl = (
        x.astype(jnp.int32)
        .T.reshape(T // 8, 8, B // LANE, LANE)
        .transpose(0, 2, 1, 3)
    )
    w2 = emb_weight
    out5 = _build(V, B, T, NC, NS)(xl, w2)
    return out5.transpose(2, 4, 0, 1, 3).reshape(B, T, DIM)
